# baseline (speedup 1.0000x reference)
import functools

import jax
import jax.numpy as jnp
from jax import lax
from jax.experimental import pallas as pl
from jax.experimental.pallas import tpu as pltpu
from jax.experimental.pallas import tpu_sc as plsc

DICT_SIZE = 100000
EMB = 128
BATCH = 16384

NUM_CORES = 2
NUM_SUBCORES = 16
NUM_WORKERS = NUM_CORES * NUM_SUBCORES
BPW = BATCH // NUM_WORKERS
CHUNK = 128
NCHUNKS = BPW // CHUNK
GROUPS = CHUNK // 16
LANES = 16
UNROLL = 16

_mesh = plsc.VectorSubcoreMesh(core_axis_name="c", subcore_axis_name="s")


@functools.partial(
    pl.kernel,
    out_type=jax.ShapeDtypeStruct((BATCH,), jnp.float32),
    mesh=_mesh,
    compiler_params=pltpu.CompilerParams(needs_layout_passes=False),
    scratch_types=[
        pltpu.VMEM((BPW,), jnp.int32),
        pltpu.VMEM((BPW,), jnp.int32),
        pltpu.VMEM((2 * CHUNK, EMB), jnp.float32),
        pltpu.VMEM((2 * CHUNK, EMB), jnp.float32),
        pltpu.VMEM((BPW,), jnp.float32),
        pltpu.SemaphoreType.DMA,
        pltpu.SemaphoreType.DMA,
    ],
)
def _word_sim(idx_hbm, tgt_hbm, table_hbm, out_hbm,
              idx_v, tgt_v, xs_v, ys_v, out_v, sem0, sem1):
    wid = lax.axis_index("s") * NUM_CORES + lax.axis_index("c")
    base = wid * BPW
    cp_i = pltpu.async_copy(idx_hbm.at[pl.ds(base, BPW)], idx_v, sem0)
    cp_t = pltpu.async_copy(tgt_hbm.at[pl.ds(base, BPW)], tgt_v, sem1)
    cp_i.wait()
    cp_t.wait()

    lane = lax.broadcasted_iota(jnp.int32, (LANES,), 0)
    sems = (sem0, sem1)

    def fire(c):
        buf = c % 2
        cpx = pltpu.async_copy(
            table_hbm.at[idx_v.at[pl.ds(c * CHUNK, CHUNK)]],
            xs_v.at[pl.ds(buf * CHUNK, CHUNK)], sems[buf])
        cpy = pltpu.async_copy(
            table_hbm.at[tgt_v.at[pl.ds(c * CHUNK, CHUNK)]],
            ys_v.at[pl.ds(buf * CHUNK, CHUNK)], sems[buf])
        return cpx, cpy

    pending = fire(0)
    for c in range(NCHUNKS):
        nxt = fire(c + 1) if c + 1 < NCHUNKS else None
        pending[0].wait()
        pending[1].wait()
        buf = c % 2

        def group_body(g, carry, c=c, buf=buf):
            rows = lane + (buf * CHUNK + g * LANES)

            def d_body(db, acc):
                for u in range(UNROLL):
                    dvec = jnp.full((LANES,), 0, jnp.int32) + (
                        db * UNROLL + u)
                    vx = plsc.load_gather(xs_v, [rows, dvec])
                    vy = plsc.load_gather(ys_v, [rows, dvec])
                    acc = acc + vx * vy
                return acc

            acc = lax.fori_loop(0, EMB // UNROLL, d_body,
                                jnp.zeros((LANES,), jnp.float32))
            out_v[pl.ds(c * CHUNK + g * LANES, LANES)] = acc
            return carry

        lax.fori_loop(0, GROUPS, group_body, 0)
        pending = nxt

    pltpu.sync_copy(out_v, out_hbm.at[pl.ds(base, BPW)])


def kernel(idx, targets, word_embs):
    return _word_sim(idx, targets, word_embs)


# speedup vs baseline: 2.4602x; 2.4602x over previous
import functools

import jax
import jax.numpy as jnp
from jax import lax
from jax.experimental import pallas as pl
from jax.experimental.pallas import tpu as pltpu
from jax.experimental.pallas import tpu_sc as plsc

DICT_SIZE = 100000
EMB = 128
BATCH = 16384

NUM_CORES = 2
NUM_SUBCORES = 16
NUM_WORKERS = NUM_CORES * NUM_SUBCORES
BPW = BATCH // NUM_WORKERS
CHUNK = 64
NCHUNKS = BPW // CHUNK
GROUPS = CHUNK // 16
LANES = 16
VPR = EMB // LANES

_mesh = plsc.VectorSubcoreMesh(core_axis_name="c", subcore_axis_name="s")


def _tree_add(parts):
    while len(parts) > 1:
        parts = [parts[i] + parts[i + 1]
                 for i in range(0, len(parts) - 1, 2)] \
            + ([parts[-1]] if len(parts) % 2 else [])
    return parts[0]


@functools.partial(
    pl.kernel,
    out_type=jax.ShapeDtypeStruct((BATCH,), jnp.float32),
    mesh=_mesh,
    compiler_params=pltpu.CompilerParams(needs_layout_passes=False),
    scratch_types=[
        pltpu.VMEM((BPW,), jnp.int32),
        pltpu.VMEM((BPW,), jnp.int32),
        pltpu.VMEM((2, CHUNK, EMB), jnp.float32),
        pltpu.VMEM((2, CHUNK, EMB), jnp.float32),
        pltpu.VMEM((2 * LANES * LANES,), jnp.float32),
        pltpu.VMEM((BPW,), jnp.float32),
        pltpu.SemaphoreType.DMA,
        pltpu.SemaphoreType.DMA,
    ],
)
def _word_sim(idx_hbm, tgt_hbm, table_hbm, out_hbm,
              idx_v, tgt_v, xs_v, ys_v, tp_v, out_v, sem0, sem1):
    wid = lax.axis_index("s") * NUM_CORES + lax.axis_index("c")
    base = wid * BPW
    cp_i = pltpu.async_copy(idx_hbm.at[pl.ds(base, BPW)], idx_v, sem0)
    cp_t = pltpu.async_copy(tgt_hbm.at[pl.ds(base, BPW)], tgt_v, sem1)
    cp_i.wait()
    cp_t.wait()

    lane = lax.broadcasted_iota(jnp.int32, (LANES,), 0)
    sems = (sem0, sem1)

    def fire(c):
        buf = c % 2
        cpx = pltpu.async_copy(
            table_hbm.at[idx_v.at[pl.ds(c * CHUNK, CHUNK)]],
            xs_v.at[buf], sems[buf])
        cpy = pltpu.async_copy(
            table_hbm.at[tgt_v.at[pl.ds(c * CHUNK, CHUNK)]],
            ys_v.at[buf], sems[buf])
        return cpx, cpy

    pending = fire(0)
    for c in range(NCHUNKS):
        nxt = fire(c + 1) if c + 1 < NCHUNKS else None
        pending[0].wait()
        pending[1].wait()
        buf = c % 2

        def group_body(g, carry, c=c, buf=buf):
            row0 = g * LANES
            h = (g % 2) * (LANES * LANES)
            def row_loads(r):
                return ([xs_v[buf, row0 + r, pl.ds(k * LANES, LANES)]
                         for k in range(VPR)],
                        [ys_v[buf, row0 + r, pl.ds(k * LANES, LANES)]
                         for k in range(VPR)])

            xv, yv = row_loads(0)
            for r in range(LANES):
                nxt_ld = row_loads(r + 1) if r + 1 < LANES else None
                p = _tree_add([xv[k] * yv[k] for k in range(VPR)])
                tp_v[pl.ds(h + r * LANES, LANES)] = p
                if nxt_ld is not None:
                    xv, yv = nxt_ld
            res = _tree_add(
                [plsc.load_gather(tp_v, [lane * LANES + (j + h)])
                 for j in range(LANES)])
            out_v[pl.ds(c * CHUNK + row0, LANES)] = res
            return carry

        lax.fori_loop(0, GROUPS, group_body, 0)
        pending = nxt

    pltpu.sync_copy(out_v, out_hbm.at[pl.ds(base, BPW)])


def kernel(idx, targets, word_embs):
    return _word_sim(idx, targets, word_embs)


# speedup vs baseline: 2.6123x; 1.0618x over previous
import functools

import jax
import jax.numpy as jnp
from jax import lax
from jax.experimental import pallas as pl
from jax.experimental.pallas import tpu as pltpu
from jax.experimental.pallas import tpu_sc as plsc

DICT_SIZE = 100000
EMB = 128
BATCH = 16384

NUM_CORES = 2
NUM_SUBCORES = 16
NUM_WORKERS = NUM_CORES * NUM_SUBCORES
BPW = BATCH // NUM_WORKERS
CHUNK = 128
NCHUNKS = BPW // CHUNK
GROUPS = CHUNK // 16
LANES = 16
VPR = EMB // LANES

_mesh = plsc.VectorSubcoreMesh(core_axis_name="c", subcore_axis_name="s")


def _tree_add(parts):
    while len(parts) > 1:
        parts = [parts[i] + parts[i + 1]
                 for i in range(0, len(parts) - 1, 2)] \
            + ([parts[-1]] if len(parts) % 2 else [])
    return parts[0]


@functools.partial(
    pl.kernel,
    out_type=jax.ShapeDtypeStruct((BATCH,), jnp.float32),
    mesh=_mesh,
    compiler_params=pltpu.CompilerParams(needs_layout_passes=False),
    scratch_types=[
        pltpu.VMEM((BPW,), jnp.int32),
        pltpu.VMEM((BPW,), jnp.int32),
        pltpu.VMEM((2, CHUNK, EMB), jnp.float32),
        pltpu.VMEM((2, CHUNK, EMB), jnp.float32),
        pltpu.VMEM((2 * LANES * LANES,), jnp.float32),
        pltpu.VMEM((BPW,), jnp.float32),
        pltpu.SemaphoreType.DMA,
        pltpu.SemaphoreType.DMA,
    ],
)
def _word_sim(idx_hbm, tgt_hbm, table_hbm, out_hbm,
              idx_v, tgt_v, xs_v, ys_v, tp_v, out_v, sem0, sem1):
    wid = lax.axis_index("s") * NUM_CORES + lax.axis_index("c")
    base = wid * BPW
    cp_i = pltpu.async_copy(idx_hbm.at[pl.ds(base, BPW)], idx_v, sem0)
    cp_t = pltpu.async_copy(tgt_hbm.at[pl.ds(base, BPW)], tgt_v, sem1)
    cp_i.wait()
    cp_t.wait()

    lane = lax.broadcasted_iota(jnp.int32, (LANES,), 0)
    sems = (sem0, sem1)

    def fire(c):
        buf = c % 2
        cpx = pltpu.async_copy(
            table_hbm.at[idx_v.at[pl.ds(c * CHUNK, CHUNK)]],
            xs_v.at[buf], sems[buf])
        cpy = pltpu.async_copy(
            table_hbm.at[tgt_v.at[pl.ds(c * CHUNK, CHUNK)]],
            ys_v.at[buf], sems[buf])
        return cpx, cpy

    pending = fire(0)
    for c in range(NCHUNKS):
        nxt = fire(c + 1) if c + 1 < NCHUNKS else None
        pending[0].wait()
        pending[1].wait()
        buf = c % 2

        def group_body(g, carry, c=c, buf=buf):
            row0 = g * LANES
            h_cur = (g % 2) * (LANES * LANES)
            h_prev = ((g + 1) % 2) * (LANES * LANES)
            g_prev = (g + GROUPS - 1) % GROUPS
            def row_loads(r):
                return ([xs_v[buf, row0 + r, pl.ds(k * LANES, LANES)]
                         for k in range(VPR)],
                        [ys_v[buf, row0 + r, pl.ds(k * LANES, LANES)]
                         for k in range(VPR)])

            xv, yv = row_loads(0)
            drains = []
            for r in range(LANES):
                nxt_ld = row_loads(r + 1) if r + 1 < LANES else None
                drains.append(
                    plsc.load_gather(tp_v, [lane * LANES + (r + h_prev)]))
                p = _tree_add([xv[k] * yv[k] for k in range(VPR)])
                tp_v[pl.ds(h_cur + r * LANES, LANES)] = p
                if nxt_ld is not None:
                    xv, yv = nxt_ld
            out_v[pl.ds(c * CHUNK + g_prev * LANES, LANES)] = \
                _tree_add(drains)
            return carry

        lax.fori_loop(0, GROUPS, group_body, 0)
        h_last = ((GROUPS - 1) % 2) * (LANES * LANES)
        res = _tree_add(
            [plsc.load_gather(tp_v, [lane * LANES + (j + h_last)])
             for j in range(LANES)])
        out_v[pl.ds(c * CHUNK + (GROUPS - 1) * LANES, LANES)] = res
        pending = nxt

    pltpu.sync_copy(out_v, out_hbm.at[pl.ds(base, BPW)])


def kernel(idx, targets, word_embs):
    return _word_sim(idx, targets, word_embs)


# speedup vs baseline: 2.6556x; 1.0166x over previous
import functools

import jax
import jax.numpy as jnp
from jax import lax
from jax.experimental import pallas as pl
from jax.experimental.pallas import tpu as pltpu
from jax.experimental.pallas import tpu_sc as plsc

DICT_SIZE = 100000
EMB = 128
BATCH = 16384

NUM_CORES = 2
NUM_SUBCORES = 16
NUM_WORKERS = NUM_CORES * NUM_SUBCORES
BPW = BATCH // NUM_WORKERS
CHUNK = 128
NCHUNKS = BPW // CHUNK
GROUPS = CHUNK // 16
LANES = 16
VPR = EMB // LANES

_mesh = plsc.VectorSubcoreMesh(core_axis_name="c", subcore_axis_name="s")


def _tree_add(parts):
    while len(parts) > 1:
        parts = [parts[i] + parts[i + 1]
                 for i in range(0, len(parts) - 1, 2)] \
            + ([parts[-1]] if len(parts) % 2 else [])
    return parts[0]


@functools.partial(
    pl.kernel,
    out_type=jax.ShapeDtypeStruct((BATCH,), jnp.float32),
    mesh=_mesh,
    compiler_params=pltpu.CompilerParams(needs_layout_passes=False),
    scratch_types=[
        pltpu.VMEM((BPW,), jnp.int32),
        pltpu.VMEM((BPW,), jnp.int32),
        pltpu.VMEM((2, CHUNK, EMB), jnp.float32),
        pltpu.VMEM((2, CHUNK, EMB), jnp.float32),
        pltpu.VMEM((2 * LANES * LANES,), jnp.float32),
        pltpu.VMEM((BPW,), jnp.float32),
        pltpu.SemaphoreType.DMA,
        pltpu.SemaphoreType.DMA,
    ],
)
def _word_sim(idx_hbm, tgt_hbm, table_hbm, out_hbm,
              idx_v, tgt_v, xs_v, ys_v, tp_v, out_v, sem0, sem1):
    wid = lax.axis_index("s") * NUM_CORES + lax.axis_index("c")
    base = wid * BPW
    cp_i = pltpu.async_copy(idx_hbm.at[pl.ds(base, BPW)], idx_v, sem0)
    cp_t = pltpu.async_copy(tgt_hbm.at[pl.ds(base, BPW)], tgt_v, sem1)
    cp_i.wait()
    cp_t.wait()

    lane = lax.broadcasted_iota(jnp.int32, (LANES,), 0)
    sems = (sem0, sem1)

    def fire(c):
        buf = c % 2
        cpx = pltpu.async_copy(
            table_hbm.at[idx_v.at[pl.ds(c * CHUNK, CHUNK)]],
            xs_v.at[buf], sems[buf])
        cpy = pltpu.async_copy(
            table_hbm.at[tgt_v.at[pl.ds(c * CHUNK, CHUNK)]],
            ys_v.at[buf], sems[buf])
        return cpx, cpy

    pending = fire(0)
    for c in range(NCHUNKS):
        nxt = fire(c + 1) if c + 1 < NCHUNKS else None
        pending[0].wait()
        pending[1].wait()
        buf = c % 2

        def group_body(g, carry, c=c, buf=buf):
            row0 = g * LANES
            h = (g % 2) * (LANES * LANES)
            def row_loads(r):
                return ([xs_v[buf, row0 + r, pl.ds(k * LANES, LANES)]
                         for k in range(VPR)],
                        [ys_v[buf, row0 + r, pl.ds(k * LANES, LANES)]
                         for k in range(VPR)])

            xv, yv = row_loads(0)
            for r in range(LANES):
                nxt_ld = row_loads(r + 1) if r + 1 < LANES else None
                p = _tree_add([xv[k] * yv[k] for k in range(VPR)])
                tp_v[pl.ds(h + r * LANES, LANES)] = p
                if nxt_ld is not None:
                    xv, yv = nxt_ld
            res = _tree_add(
                [plsc.load_gather(tp_v, [lane * LANES + (j + h)])
                 for j in range(LANES)])
            out_v[pl.ds(c * CHUNK + row0, LANES)] = res
            return carry

        lax.fori_loop(0, GROUPS, group_body, 0)
        pending = nxt

    pltpu.sync_copy(out_v, out_hbm.at[pl.ds(base, BPW)])


def kernel(idx, targets, word_embs):
    return _word_sim(idx, targets, word_embs)
